# Initial kernel scaffold; baseline (speedup 1.0000x reference)
import jax
import jax.numpy as jnp
from jax.experimental import pallas as pl


def kernel(feats, labels):
    raise NotImplementedError("write your pallas kernel here")


# speedup vs baseline: 21.3062x; 21.3062x over previous
import functools

import jax
import jax.numpy as jnp
from jax import lax
from jax.experimental import pallas as pl
from jax.experimental.pallas import tpu as pltpu

N = 4096
D = 1024
BR = 128
SENTINEL = 0x7F7FFFFE


def _normalize_body(x_ref, xn_ref, sq_ref):
    x = x_ref[...]
    s0 = jnp.sum(x * x, axis=1, keepdims=True)
    norm = jnp.maximum(jnp.sqrt(s0), 1e-12)
    xn = x / norm
    xn_ref[...] = xn
    sqn = jnp.sum(xn * xn, axis=1)
    sq_ref[...] = sqn.reshape(1, -1)


def _bitonic_sort_rows(a):
    n = a.shape[-1]
    nbits = n.bit_length() - 1
    nstages = nbits * (nbits + 1) // 2
    col = lax.broadcasted_iota(jnp.int32, (1, n), 1)

    def stage(_, carry):
        a, p, q = carry
        s = jnp.left_shift(jnp.int32(1), q)
        low = ((col >> q) & 1) == 0
        partner = jnp.where(low, pltpu.roll(a, n - s, 1), pltpu.roll(a, s, 1))
        mn = jnp.minimum(a, partner)
        mx = jnp.maximum(a, partner)
        asc = ((col >> (p + 1)) & 1) == 0
        a = jnp.where(asc == low, mn, mx)
        last = q == 0
        p2 = jnp.where(last, p + 1, p)
        q2 = jnp.where(last, p + 1, q - 1)
        return a, p2, q2

    a, _, _ = lax.fori_loop(
        0, nstages, stage, (a, jnp.int32(0), jnp.int32(0))
    )
    return a


def _main_body(xn_ref, xq_ref, sq_ref, lab_ref, labq_ref, ap_ref, r1_ref, val_ref):
    i = pl.program_id(0)
    xn = xn_ref[...]
    xq = xq_ref[...]
    g = lax.dot_general(
        xq, xn, (((1,), (1,)), ((), ())), preferred_element_type=jnp.float32
    )
    sq_all = sq_ref[...]
    sq_q = jnp.sum(xq * xq, axis=1, keepdims=True)
    d = jnp.maximum(sq_q + sq_all - 2.0 * g, 0.0)

    labels = lab_ref[...]
    lab_q = labq_ref[...]
    col = lax.broadcasted_iota(jnp.int32, (BR, N), 1)
    row = i * BR + lax.broadcasted_iota(jnp.int32, (BR, N), 0)
    is_self = col == row
    match = (labels == lab_q) & jnp.logical_not(is_self)

    bits = lax.bitcast_convert_type(d, jnp.int32)
    key = (bits & jnp.int32(~1)) | match.astype(jnp.int32)
    key = jnp.where(is_self, SENTINEL, key)

    skey = _bitonic_sort_rows(key)

    m = (skey & 1).astype(jnp.float32)
    cols = lax.broadcasted_iota(jnp.int32, (1, N), 1)
    ch = m
    s = 1
    while s < N:
        ch = ch + jnp.where(cols >= s, pltpu.roll(ch, s, 1), 0.0)
        s *= 2
    inv_pos = 1.0 / (
        1.0 + lax.broadcasted_iota(jnp.int32, (1, N), 1).astype(jnp.float32)
    )
    prec_sum = jnp.sum(m * ch * inv_pos, axis=1)
    num_rel = ch[:, N - 1]
    valid = (num_rel > 0).astype(jnp.float32)
    ap = prec_sum / jnp.maximum(num_rel, 1.0) * valid
    r1 = m[:, 0] * valid

    ap_ref[...] = ap.reshape(1, BR)
    r1_ref[...] = r1.reshape(1, BR)
    val_ref[...] = valid.reshape(1, BR)


def _finish_body(ap_ref, r1_ref, val_ref, out_ref):
    s_ap = jnp.sum(ap_ref[...])
    s_r1 = jnp.sum(r1_ref[...])
    nq = jnp.maximum(jnp.sum(val_ref[...]), 1.0)
    cid = lax.broadcasted_iota(jnp.int32, (1, 2), 1)
    out_ref[...] = jnp.where(cid == 0, s_r1 / nq, s_ap / nq)


@jax.jit
def kernel(feats, labels):
    xn, sq = pl.pallas_call(
        _normalize_body,
        grid=(N // 512,),
        in_specs=[pl.BlockSpec((512, D), lambda i: (i, 0))],
        out_specs=[
            pl.BlockSpec((512, D), lambda i: (i, 0)),
            pl.BlockSpec((1, 512), lambda i: (0, i)),
        ],
        out_shape=[
            jax.ShapeDtypeStruct((N, D), jnp.float32),
            jax.ShapeDtypeStruct((1, N), jnp.float32),
        ],
    )(feats)

    lab2 = labels.reshape(1, N)
    labT = labels.reshape(N, 1)
    ap, r1, val = pl.pallas_call(
        _main_body,
        grid=(N // BR,),
        in_specs=[
            pl.BlockSpec((N, D), lambda i: (0, 0)),
            pl.BlockSpec((BR, D), lambda i: (i, 0)),
            pl.BlockSpec((1, N), lambda i: (0, 0)),
            pl.BlockSpec((1, N), lambda i: (0, 0)),
            pl.BlockSpec((BR, 1), lambda i: (i, 0)),
        ],
        out_specs=[
            pl.BlockSpec((1, BR), lambda i: (0, i)),
            pl.BlockSpec((1, BR), lambda i: (0, i)),
            pl.BlockSpec((1, BR), lambda i: (0, i)),
        ],
        out_shape=[
            jax.ShapeDtypeStruct((1, N), jnp.float32),
            jax.ShapeDtypeStruct((1, N), jnp.float32),
            jax.ShapeDtypeStruct((1, N), jnp.float32),
        ],
    )(xn, xn, sq, lab2, labT)

    out = pl.pallas_call(
        _finish_body,
        out_shape=jax.ShapeDtypeStruct((1, 2), jnp.float32),
    )(ap, r1, val)
    return out.reshape(2)


# speedup vs baseline: 34.9222x; 1.6391x over previous
import jax
import jax.numpy as jnp
from jax import lax
from jax.experimental import pallas as pl
from jax.experimental.pallas import tpu as pltpu

N = 4096
D = 1024
BK = 256
BS = 8
SENTINEL = 0x7F7FFFFE


def _normalize_body(x_ref, xn_ref, sq_ref):
    x = x_ref[...]
    s0 = jnp.sum(x * x, axis=1, keepdims=True)
    norm = jnp.maximum(jnp.sqrt(s0), 1e-12)
    xn = x / norm
    xn_ref[...] = xn
    sqn = jnp.sum(xn * xn, axis=1)
    sq_ref[...] = sqn.reshape(1, -1)


def _keys_body(xn_ref, xq_ref, sq_ref, lab_ref, labq_ref, key_ref):
    i = pl.program_id(0)
    xn = xn_ref[...]
    xq = xq_ref[...]
    g = lax.dot_general(
        xq, xn, (((1,), (1,)), ((), ())), preferred_element_type=jnp.float32
    )
    sq_all = sq_ref[...]
    sq_q = jnp.sum(xq * xq, axis=1, keepdims=True)
    d = jnp.maximum(sq_q + sq_all - 2.0 * g, 0.0)

    labels = lab_ref[...]
    lab_q = labq_ref[...]
    col = lax.broadcasted_iota(jnp.int32, (BK, N), 1)
    row = i * BK + lax.broadcasted_iota(jnp.int32, (BK, N), 0)
    is_self = col == row
    match = (labels == lab_q) & jnp.logical_not(is_self)

    bits = lax.bitcast_convert_type(d, jnp.int32)
    key = (bits & jnp.int32(~1)) | match.astype(jnp.int32)
    key_ref[...] = jnp.where(is_self, SENTINEL, key)


def _bitonic_sort_rows(a):
    n = a.shape[-1]
    nbits = n.bit_length() - 1
    col = lax.broadcasted_iota(jnp.int32, (1, n), 1)
    for p in range(nbits):
        for q in range(p, -1, -1):
            s = 1 << q
            low = ((col >> q) & 1) == 0
            partner = jnp.where(low, pltpu.roll(a, n - s, 1), pltpu.roll(a, s, 1))
            mn = jnp.minimum(a, partner)
            mx = jnp.maximum(a, partner)
            if p == nbits - 1:
                keep_min = low
            else:
                asc = ((col >> (p + 1)) & 1) == 0
                keep_min = asc == low
            a = jnp.where(keep_min, mn, mx)
    return a


def _sort_body(key_ref, ap_ref, r1_ref, val_ref):
    skey = _bitonic_sort_rows(key_ref[...])

    m = (skey & 1).astype(jnp.float32)
    cols = lax.broadcasted_iota(jnp.int32, (1, N), 1)
    ch = m
    s = 1
    while s < N:
        ch = ch + jnp.where(cols >= s, pltpu.roll(ch, s, 1), 0.0)
        s *= 2
    inv_pos = 1.0 / (1.0 + cols.astype(jnp.float32))
    prec_sum = jnp.sum(m * ch * inv_pos, axis=1)
    num_rel = ch[:, N - 1]
    valid = (num_rel > 0).astype(jnp.float32)
    ap = prec_sum / jnp.maximum(num_rel, 1.0) * valid
    r1 = m[:, 0] * valid

    ap_ref[...] = ap.reshape(1, 1, BS)
    r1_ref[...] = r1.reshape(1, 1, BS)
    val_ref[...] = valid.reshape(1, 1, BS)


def _finish_body(ap_ref, r1_ref, val_ref, out_ref):
    s_ap = jnp.sum(ap_ref[...])
    s_r1 = jnp.sum(r1_ref[...])
    nq = jnp.maximum(jnp.sum(val_ref[...]), 1.0)
    cid = lax.broadcasted_iota(jnp.int32, (1, 2), 1)
    out_ref[...] = jnp.where(cid == 0, s_r1 / nq, s_ap / nq)


@jax.jit
def kernel(feats, labels):
    xn, sq = pl.pallas_call(
        _normalize_body,
        grid=(N // 512,),
        in_specs=[pl.BlockSpec((512, D), lambda i: (i, 0))],
        out_specs=[
            pl.BlockSpec((512, D), lambda i: (i, 0)),
            pl.BlockSpec((1, 512), lambda i: (0, i)),
        ],
        out_shape=[
            jax.ShapeDtypeStruct((N, D), jnp.float32),
            jax.ShapeDtypeStruct((1, N), jnp.float32),
        ],
    )(feats)

    lab2 = labels.reshape(1, N)
    labT = labels.reshape(N, 1)
    keys = pl.pallas_call(
        _keys_body,
        grid=(N // BK,),
        in_specs=[
            pl.BlockSpec((N, D), lambda i: (0, 0)),
            pl.BlockSpec((BK, D), lambda i: (i, 0)),
            pl.BlockSpec((1, N), lambda i: (0, 0)),
            pl.BlockSpec((1, N), lambda i: (0, 0)),
            pl.BlockSpec((BK, 1), lambda i: (i, 0)),
        ],
        out_specs=pl.BlockSpec((BK, N), lambda i: (i, 0)),
        out_shape=jax.ShapeDtypeStruct((N, N), jnp.int32),
    )(xn, xn, sq, lab2, labT)

    ap, r1, val = pl.pallas_call(
        _sort_body,
        grid=(N // BS,),
        in_specs=[pl.BlockSpec((BS, N), lambda i: (i, 0))],
        out_specs=[
            pl.BlockSpec((1, 1, BS), lambda i: (i, 0, 0)),
            pl.BlockSpec((1, 1, BS), lambda i: (i, 0, 0)),
            pl.BlockSpec((1, 1, BS), lambda i: (i, 0, 0)),
        ],
        out_shape=[
            jax.ShapeDtypeStruct((N // BS, 1, BS), jnp.float32),
            jax.ShapeDtypeStruct((N // BS, 1, BS), jnp.float32),
            jax.ShapeDtypeStruct((N // BS, 1, BS), jnp.float32),
        ],
    )(keys)

    out = pl.pallas_call(
        _finish_body,
        out_shape=jax.ShapeDtypeStruct((1, 2), jnp.float32),
    )(ap.reshape(1, N), r1.reshape(1, N), val.reshape(1, N))
    return out.reshape(2)
